# Initial kernel scaffold; baseline (speedup 1.0000x reference)
import jax
import jax.numpy as jnp
from jax.experimental import pallas as pl


def kernel(inputs, domain_indicator, gamma, beta):
    raise NotImplementedError("write your pallas kernel here")


# speedup vs baseline: 2.5854x; 2.5854x over previous
import functools

import jax
import jax.numpy as jnp
from jax import lax
from jax.experimental import pallas as pl
from jax.experimental.pallas import tpu as pltpu
from jax.experimental.pallas import tpu_sc as plsc

NUM_DOMAINS = 4
EPS = 1e-3
B, D = 16384, 128
L = 16
NC, NS = 2, 16
NW = NC * NS
RPW = B // NW
CH = 4
CR = RPW // CH
ACC = 9 * D

_mesh = plsc.VectorSubcoreMesh(core_axis_name="c", subcore_axis_name="s")
_params = pltpu.CompilerParams(needs_layout_passes=False)


def _iota16():
    return lax.broadcasted_iota(jnp.int32, (L,), 0)


def _rsqrt(x):
    i = plsc.bitcast(x, jnp.int32)
    y = plsc.bitcast(jnp.int32(0x5F3759DF) - (i >> 1), jnp.float32)
    for _ in range(3):
        y = y * (1.5 - 0.5 * x * y * y)
    return y


@functools.partial(
    pl.kernel,
    out_type=(
        jax.ShapeDtypeStruct((NW * ACC,), jnp.float32),
        jax.ShapeDtypeStruct((B,), jnp.int32),
    ),
    mesh=_mesh,
    compiler_params=_params,
    scratch_types=[
        pltpu.VMEM((RPW * D,), jnp.float32),
        pltpu.VMEM((RPW * NUM_DOMAINS,), jnp.float32),
        pltpu.VMEM((RPW,), jnp.int32),
        pltpu.VMEM((ACC,), jnp.float32),
        pltpu.SemaphoreType.DMA,
        pltpu.SemaphoreType.DMA,
        pltpu.SemaphoreType.DMA,
        pltpu.SemaphoreType.DMA,
    ],
)
def _stats_kernel(x_hbm, di_hbm, parts_hbm, didx_hbm,
                  data_v, di_v, didx_v, acc_v, s0, s1, s2, s3):
    wid = lax.axis_index("s") * NC + lax.axis_index("c")
    row0 = wid * RPW
    iota = _iota16()
    sems = [s0, s1, s2, s3]

    cps = []
    for c in range(CH):
        cps.append(pltpu.async_copy(
            x_hbm.at[pl.ds((row0 + c * CR) * D, CR * D)],
            data_v.at[pl.ds(c * CR * D, CR * D)],
            sems[c]))
    pltpu.sync_copy(di_hbm.at[pl.ds(row0 * NUM_DOMAINS, RPW * NUM_DOMAINS)],
                    di_v)

    def zero_body(q, _):
        acc_v[pl.ds(q * L, L)] = jnp.zeros((L,), jnp.float32)
        return 0

    lax.fori_loop(0, ACC // L, zero_body, 0)

    def didx_body(t, _):
        idx0 = t * (L * NUM_DOMAINS) + iota * NUM_DOMAINS
        best = plsc.load_gather(di_v, [idx0])
        bidx = jnp.zeros((L,), jnp.int32)
        for c in range(1, NUM_DOMAINS):
            v = plsc.load_gather(di_v, [idx0 + c])
            take = v > best
            best = jnp.where(take, v, best)
            bidx = jnp.where(take, jnp.full((L,), c, jnp.int32), bidx)
        didx_v[pl.ds(t * L, L)] = bidx
        return 0

    lax.fori_loop(0, RPW // L, didx_body, 0)

    ones = jnp.ones((L,), jnp.float32)

    def acc_body(r, _):
        d_b = plsc.load_gather(didx_v, [jnp.zeros((L,), jnp.int32) + r])
        col = d_b * D + iota
        for j in range(D // L):
            v = data_v[pl.ds(r * D + j * L, L)]
            plsc.addupdate_scatter(acc_v, [col + j * L], v)
            plsc.addupdate_scatter(acc_v, [col + j * L + 4 * D], v * v)
        plsc.addupdate_scatter(acc_v, [8 * D + d_b * L + iota], ones)
        return 0

    for c in range(CH):
        cps[c].wait()
        lax.fori_loop(c * CR, (c + 1) * CR, acc_body, 0)

    pltpu.sync_copy(acc_v, parts_hbm.at[pl.ds(wid * ACC, ACC)])
    pltpu.sync_copy(didx_v, didx_hbm.at[pl.ds(row0, RPW)])


@functools.partial(
    pl.kernel,
    out_type=jax.ShapeDtypeStruct((B * D,), jnp.float32),
    mesh=_mesh,
    compiler_params=_params,
    scratch_types=[
        pltpu.VMEM((RPW * D,), jnp.float32),
        pltpu.VMEM((RPW,), jnp.int32),
        pltpu.VMEM((NW * ACC,), jnp.float32),
        pltpu.VMEM((ACC,), jnp.float32),
        pltpu.VMEM((8 * D,), jnp.float32),
        pltpu.VMEM((4 * D,), jnp.float32),
        pltpu.VMEM((4 * D,), jnp.float32),
        pltpu.VMEM_SHARED((8 * D,), jnp.float32),
        pltpu.SemaphoreType.DMA,
        pltpu.SemaphoreType.DMA,
        pltpu.SemaphoreType.DMA,
        pltpu.SemaphoreType.DMA,
        pltpu.SemaphoreType.DMA,
        pltpu.SemaphoreType.DMA,
        pltpu.SemaphoreType.DMA,
        pltpu.SemaphoreType.DMA,
    ],
)
def _norm_kernel(x_hbm, didx_hbm, parts_hbm, g_hbm, b_hbm, out_hbm,
                 data_v, didx_v, parts_v, tot_v, so_v, g_v, b_v, so_sh,
                 i0, i1, i2, i3, o0, o1, o2, o3):
    sid = lax.axis_index("s")
    wid = sid * NC + lax.axis_index("c")
    row0 = wid * RPW
    iota = _iota16()
    isems = [i0, i1, i2, i3]
    osems = [o0, o1, o2, o3]

    cps = []
    for c in range(CH):
        cps.append(pltpu.async_copy(
            x_hbm.at[pl.ds((row0 + c * CR) * D, CR * D)],
            data_v.at[pl.ds(c * CR * D, CR * D)],
            isems[c]))
    pltpu.sync_copy(didx_hbm.at[pl.ds(row0, RPW)], didx_v)

    @pl.when(sid == 0)
    def _():
        pltpu.sync_copy(parts_hbm, parts_v)
        pltpu.sync_copy(g_hbm, g_v)
        pltpu.sync_copy(b_hbm, b_v)

        def red_body(q, _):
            def wsum(w, a):
                return a + parts_v[pl.ds(w * ACC + q * L, L)]

            tot_v[pl.ds(q * L, L)] = lax.fori_loop(
                0, NW, wsum, jnp.zeros((L,), jnp.float32))
            return 0

        lax.fori_loop(0, ACC // L, red_body, 0)

        for d in range(NUM_DOMAINS):
            cnt = tot_v[pl.ds(8 * D + d * L, L)]
            safe = jnp.maximum(cnt, 1.0)
            for j in range(D // L):
                off = d * D + j * L
                sm = tot_v[pl.ds(off, L)]
                sq = tot_v[pl.ds(4 * D + off, L)]
                mean = sm / safe
                var = jnp.maximum(sq / safe - mean * mean, 0.0)
                s = g_v[pl.ds(off, L)] * _rsqrt(var + EPS)
                so_v[pl.ds(off, L)] = s
                so_v[pl.ds(4 * D + off, L)] = b_v[pl.ds(off, L)] - mean * s

        pltpu.sync_copy(so_v, so_sh)

    plsc.subcore_barrier()
    pltpu.sync_copy(so_sh, so_v)

    def norm_body(r, _):
        d_b = plsc.load_gather(didx_v, [jnp.zeros((L,), jnp.int32) + r])
        col = d_b * D + iota
        for j in range(D // L):
            v = data_v[pl.ds(r * D + j * L, L)]
            s = plsc.load_gather(so_v, [col + j * L])
            o = plsc.load_gather(so_v, [col + j * L + 4 * D])
            data_v[pl.ds(r * D + j * L, L)] = v * s + o
        return 0

    ocps = []
    for c in range(CH):
        cps[c].wait()
        lax.fori_loop(c * CR, (c + 1) * CR, norm_body, 0)
        ocps.append(pltpu.async_copy(
            data_v.at[pl.ds(c * CR * D, CR * D)],
            out_hbm.at[pl.ds((row0 + c * CR) * D, CR * D)],
            osems[c]))
    for c in range(CH):
        ocps[c].wait()


def kernel(inputs, domain_indicator, gamma, beta):
    x = inputs.reshape(-1)
    di = domain_indicator.reshape(-1)
    parts, didx = _stats_kernel(x, di)
    out = _norm_kernel(x, didx, parts,
                       gamma[:, :D].reshape(-1), beta[:, :D].reshape(-1))
    return out.reshape(B, D)


# speedup vs baseline: 2.8106x; 1.0871x over previous
import functools

import jax
import jax.numpy as jnp
from jax import lax
from jax.experimental import pallas as pl
from jax.experimental.pallas import tpu as pltpu
from jax.experimental.pallas import tpu_sc as plsc

NUM_DOMAINS = 4
EPS = 1e-3
B, D = 16384, 128
L = 16
NC, NS = 2, 16
NW = NC * NS
RPW = B // NW
CH = 4
CR = RPW // CH
ACC = 9 * D

_mesh = plsc.VectorSubcoreMesh(core_axis_name="c", subcore_axis_name="s")
_params = pltpu.CompilerParams(needs_layout_passes=False)


def _iota16():
    return lax.broadcasted_iota(jnp.int32, (L,), 0)


def _rsqrt(x):
    i = plsc.bitcast(x, jnp.int32)
    y = plsc.bitcast(jnp.int32(0x5F3759DF) - (i >> 1), jnp.float32)
    for _ in range(3):
        y = y * (1.5 - 0.5 * x * y * y)
    return y


@functools.partial(
    pl.kernel,
    out_type=(
        jax.ShapeDtypeStruct((NW * ACC,), jnp.float32),
        jax.ShapeDtypeStruct((B,), jnp.int32),
    ),
    mesh=_mesh,
    compiler_params=_params,
    scratch_types=[
        pltpu.VMEM((RPW * D,), jnp.float32),
        pltpu.VMEM((RPW * NUM_DOMAINS,), jnp.float32),
        pltpu.VMEM((RPW,), jnp.int32),
        pltpu.VMEM((ACC,), jnp.float32),
        pltpu.SemaphoreType.DMA,
        pltpu.SemaphoreType.DMA,
        pltpu.SemaphoreType.DMA,
        pltpu.SemaphoreType.DMA,
    ],
)
def _stats_kernel(x_hbm, di_hbm, parts_hbm, didx_hbm,
                  data_v, di_v, didx_v, acc_v, s0, s1, s2, s3):
    wid = lax.axis_index("s") * NC + lax.axis_index("c")
    row0 = wid * RPW
    iota = _iota16()
    sems = [s0, s1, s2, s3]

    cps = []
    for c in range(CH):
        cps.append(pltpu.async_copy(
            x_hbm.at[pl.ds((row0 + c * CR) * D, CR * D)],
            data_v.at[pl.ds(c * CR * D, CR * D)],
            sems[c]))
    pltpu.sync_copy(di_hbm.at[pl.ds(row0 * NUM_DOMAINS, RPW * NUM_DOMAINS)],
                    di_v)

    def zero_body(q, _):
        acc_v[pl.ds(q * L, L)] = jnp.zeros((L,), jnp.float32)
        return 0

    lax.fori_loop(0, ACC // L, zero_body, 0)

    def didx_body(t, _):
        idx0 = t * (L * NUM_DOMAINS) + iota * NUM_DOMAINS
        best = plsc.load_gather(di_v, [idx0])
        bidx = jnp.zeros((L,), jnp.int32)
        for c in range(1, NUM_DOMAINS):
            v = plsc.load_gather(di_v, [idx0 + c])
            take = v > best
            best = jnp.where(take, v, best)
            bidx = jnp.where(take, jnp.full((L,), c, jnp.int32), bidx)
        didx_v[pl.ds(t * L, L)] = bidx
        return 0

    lax.fori_loop(0, RPW // L, didx_body, 0, unroll=2)

    ones = jnp.ones((L,), jnp.float32)

    def acc_body(r, _):
        d_b = plsc.load_gather(didx_v, [jnp.zeros((L,), jnp.int32) + r])
        col = d_b * D + iota
        for j in range(D // L):
            v = data_v[pl.ds(r * D + j * L, L)]
            plsc.addupdate_scatter(acc_v, [col + j * L], v)
            plsc.addupdate_scatter(acc_v, [col + j * L + 4 * D], v * v)
        plsc.addupdate_scatter(acc_v, [8 * D + d_b * L + iota], ones)
        return 0

    for c in range(CH):
        cps[c].wait()
        lax.fori_loop(c * CR, (c + 1) * CR, acc_body, 0, unroll=4)

    pltpu.sync_copy(acc_v, parts_hbm.at[pl.ds(wid * ACC, ACC)])
    pltpu.sync_copy(didx_v, didx_hbm.at[pl.ds(row0, RPW)])


@functools.partial(
    pl.kernel,
    out_type=jax.ShapeDtypeStruct((B * D,), jnp.float32),
    mesh=_mesh,
    compiler_params=_params,
    scratch_types=[
        pltpu.VMEM((RPW * D,), jnp.float32),
        pltpu.VMEM((RPW,), jnp.int32),
        pltpu.VMEM((NW * ACC,), jnp.float32),
        pltpu.VMEM((ACC,), jnp.float32),
        pltpu.VMEM((8 * D,), jnp.float32),
        pltpu.VMEM((4 * D,), jnp.float32),
        pltpu.VMEM((4 * D,), jnp.float32),
        pltpu.VMEM_SHARED((8 * D,), jnp.float32),
        pltpu.SemaphoreType.DMA,
        pltpu.SemaphoreType.DMA,
        pltpu.SemaphoreType.DMA,
        pltpu.SemaphoreType.DMA,
        pltpu.SemaphoreType.DMA,
        pltpu.SemaphoreType.DMA,
        pltpu.SemaphoreType.DMA,
        pltpu.SemaphoreType.DMA,
    ],
)
def _norm_kernel(x_hbm, didx_hbm, parts_hbm, g_hbm, b_hbm, out_hbm,
                 data_v, didx_v, parts_v, tot_v, so_v, g_v, b_v, so_sh,
                 i0, i1, i2, i3, o0, o1, o2, o3):
    sid = lax.axis_index("s")
    wid = sid * NC + lax.axis_index("c")
    row0 = wid * RPW
    iota = _iota16()
    isems = [i0, i1, i2, i3]
    osems = [o0, o1, o2, o3]

    cps = []
    for c in range(CH):
        cps.append(pltpu.async_copy(
            x_hbm.at[pl.ds((row0 + c * CR) * D, CR * D)],
            data_v.at[pl.ds(c * CR * D, CR * D)],
            isems[c]))
    pltpu.sync_copy(didx_hbm.at[pl.ds(row0, RPW)], didx_v)

    @pl.when(sid == 0)
    def _():
        pltpu.sync_copy(parts_hbm, parts_v)
        pltpu.sync_copy(g_hbm, g_v)
        pltpu.sync_copy(b_hbm, b_v)

        def red_body(q, _):
            acc = [parts_v[pl.ds(w * ACC + q * L, L)] for w in range(NW)]
            while len(acc) > 1:
                acc = [a + b for a, b in zip(acc[::2], acc[1::2])]
            tot_v[pl.ds(q * L, L)] = acc[0]
            return 0

        lax.fori_loop(0, ACC // L, red_body, 0, unroll=2)

        for d in range(NUM_DOMAINS):
            cnt = tot_v[pl.ds(8 * D + d * L, L)]
            safe = jnp.maximum(cnt, 1.0)
            for j in range(D // L):
                off = d * D + j * L
                sm = tot_v[pl.ds(off, L)]
                sq = tot_v[pl.ds(4 * D + off, L)]
                mean = sm / safe
                var = jnp.maximum(sq / safe - mean * mean, 0.0)
                s = g_v[pl.ds(off, L)] * _rsqrt(var + EPS)
                so_v[pl.ds(off, L)] = s
                so_v[pl.ds(4 * D + off, L)] = b_v[pl.ds(off, L)] - mean * s

        pltpu.sync_copy(so_v, so_sh)

    plsc.subcore_barrier()
    pltpu.sync_copy(so_sh, so_v)

    def norm_body(r, _):
        d_b = plsc.load_gather(didx_v, [jnp.zeros((L,), jnp.int32) + r])
        col = d_b * D + iota
        for j in range(D // L):
            v = data_v[pl.ds(r * D + j * L, L)]
            s = plsc.load_gather(so_v, [col + j * L])
            o = plsc.load_gather(so_v, [col + j * L + 4 * D])
            data_v[pl.ds(r * D + j * L, L)] = v * s + o
        return 0

    ocps = []
    for c in range(CH):
        cps[c].wait()
        lax.fori_loop(c * CR, (c + 1) * CR, norm_body, 0, unroll=4)
        ocps.append(pltpu.async_copy(
            data_v.at[pl.ds(c * CR * D, CR * D)],
            out_hbm.at[pl.ds((row0 + c * CR) * D, CR * D)],
            osems[c]))
    for c in range(CH):
        ocps[c].wait()


def kernel(inputs, domain_indicator, gamma, beta):
    x = inputs.reshape(-1)
    di = domain_indicator.reshape(-1)
    parts, didx = _stats_kernel(x, di)
    out = _norm_kernel(x, didx, parts,
                       gamma[:, :D].reshape(-1), beta[:, :D].reshape(-1))
    return out.reshape(B, D)


# speedup vs baseline: 4.0487x; 1.4405x over previous
import functools

import jax
import jax.numpy as jnp
from jax import lax
from jax.experimental import pallas as pl
from jax.experimental.pallas import tpu as pltpu
from jax.experimental.pallas import tpu_sc as plsc

NUM_DOMAINS = 4
EPS = 1e-3
B, D = 16384, 128
L = 16
NC, NS = 2, 16
NW = NC * NS
RPW = B // NW
CH = 4
CR = RPW // CH
ACC = 9 * D

_mesh = plsc.VectorSubcoreMesh(core_axis_name="c", subcore_axis_name="s")
_params = pltpu.CompilerParams(needs_layout_passes=False)


def _iota16():
    return lax.broadcasted_iota(jnp.int32, (L,), 0)


def _rsqrt(x):
    i = plsc.bitcast(x, jnp.int32)
    y = plsc.bitcast(jnp.int32(0x5F3759DF) - (i >> 1), jnp.float32)
    for _ in range(3):
        y = y * (1.5 - 0.5 * x * y * y)
    return y


@functools.partial(
    pl.kernel,
    out_type=(
        jax.ShapeDtypeStruct((NW * ACC,), jnp.float32),
        jax.ShapeDtypeStruct((B,), jnp.int32),
    ),
    mesh=_mesh,
    compiler_params=_params,
    scratch_types=[
        pltpu.VMEM((RPW * D,), jnp.float32),
        pltpu.VMEM((RPW * NUM_DOMAINS,), jnp.float32),
        pltpu.VMEM((RPW,), jnp.int32),
        pltpu.VMEM((ACC,), jnp.float32),
        pltpu.SemaphoreType.DMA,
        pltpu.SemaphoreType.DMA,
        pltpu.SemaphoreType.DMA,
        pltpu.SemaphoreType.DMA,
    ],
)
def _stats_kernel(x_hbm, di_hbm, parts_hbm, didx_hbm,
                  data_v, di_v, didx_v, acc_v, s0, s1, s2, s3):
    wid = lax.axis_index("s") * NC + lax.axis_index("c")
    row0 = wid * RPW
    iota = _iota16()
    sems = [s0, s1, s2, s3]

    cps = []
    for c in range(CH):
        cps.append(pltpu.async_copy(
            x_hbm.at[pl.ds((row0 + c * CR) * D, CR * D)],
            data_v.at[pl.ds(c * CR * D, CR * D)],
            sems[c]))
    pltpu.sync_copy(di_hbm.at[pl.ds(row0 * NUM_DOMAINS, RPW * NUM_DOMAINS)],
                    di_v)

    def zero_body(q, _):
        acc_v[pl.ds(q * L, L)] = jnp.zeros((L,), jnp.float32)
        return 0

    lax.fori_loop(0, ACC // L, zero_body, 0)

    def didx_body(t, _):
        idx0 = t * (L * NUM_DOMAINS) + iota * NUM_DOMAINS
        best = plsc.load_gather(di_v, [idx0])
        bidx = jnp.zeros((L,), jnp.int32)
        for c in range(1, NUM_DOMAINS):
            v = plsc.load_gather(di_v, [idx0 + c])
            take = v > best
            best = jnp.where(take, v, best)
            bidx = jnp.where(take, jnp.full((L,), c, jnp.int32), bidx)
        didx_v[pl.ds(t * L, L)] = bidx
        return 0

    lax.fori_loop(0, RPW // L, didx_body, 0, unroll=2)

    ones = jnp.ones((L,), jnp.float32)

    def acc_body(r, _):
        d_b = plsc.load_gather(didx_v, [jnp.zeros((L,), jnp.int32) + r])
        col = d_b * D + iota
        vs = [data_v[pl.ds(r * D + j * L, L)] for j in range(D // L)]
        sqs = [v * v for v in vs]
        for j in range(D // L):
            plsc.addupdate_scatter(acc_v, [col + j * L], vs[j])
        for j in range(D // L):
            plsc.addupdate_scatter(acc_v, [col + j * L + 4 * D], sqs[j])
        plsc.addupdate_scatter(acc_v, [8 * D + d_b * L + iota], ones)
        return 0

    for c in range(CH):
        cps[c].wait()
        lax.fori_loop(c * CR, (c + 1) * CR, acc_body, 0, unroll=4)

    pltpu.sync_copy(acc_v, parts_hbm.at[pl.ds(wid * ACC, ACC)])
    pltpu.sync_copy(didx_v, didx_hbm.at[pl.ds(row0, RPW)])


@functools.partial(
    pl.kernel,
    out_type=jax.ShapeDtypeStruct((B * D,), jnp.float32),
    mesh=_mesh,
    compiler_params=_params,
    scratch_types=[
        pltpu.VMEM((RPW * D,), jnp.float32),
        pltpu.VMEM((RPW,), jnp.int32),
        pltpu.VMEM((NW * ACC,), jnp.float32),
        pltpu.VMEM((ACC,), jnp.float32),
        pltpu.VMEM((8 * D,), jnp.float32),
        pltpu.VMEM((4 * D,), jnp.float32),
        pltpu.VMEM((4 * D,), jnp.float32),
        pltpu.VMEM_SHARED((8 * D,), jnp.float32),
        pltpu.SemaphoreType.DMA,
        pltpu.SemaphoreType.DMA,
        pltpu.SemaphoreType.DMA,
        pltpu.SemaphoreType.DMA,
        pltpu.SemaphoreType.DMA,
        pltpu.SemaphoreType.DMA,
        pltpu.SemaphoreType.DMA,
        pltpu.SemaphoreType.DMA,
    ],
)
def _norm_kernel(x_hbm, didx_hbm, parts_hbm, g_hbm, b_hbm, out_hbm,
                 data_v, didx_v, parts_v, tot_v, so_v, g_v, b_v, so_sh,
                 i0, i1, i2, i3, o0, o1, o2, o3):
    sid = lax.axis_index("s")
    wid = sid * NC + lax.axis_index("c")
    row0 = wid * RPW
    iota = _iota16()
    isems = [i0, i1, i2, i3]
    osems = [o0, o1, o2, o3]

    cps = []
    for c in range(CH):
        cps.append(pltpu.async_copy(
            x_hbm.at[pl.ds((row0 + c * CR) * D, CR * D)],
            data_v.at[pl.ds(c * CR * D, CR * D)],
            isems[c]))
    pltpu.sync_copy(didx_hbm.at[pl.ds(row0, RPW)], didx_v)

    @pl.when(sid == 0)
    def _():
        pltpu.sync_copy(parts_hbm, parts_v)
        pltpu.sync_copy(g_hbm, g_v)
        pltpu.sync_copy(b_hbm, b_v)

        def red_body(q, _):
            acc = [parts_v[pl.ds(w * ACC + q * L, L)] for w in range(NW)]
            while len(acc) > 1:
                acc = [a + b for a, b in zip(acc[::2], acc[1::2])]
            tot_v[pl.ds(q * L, L)] = acc[0]
            return 0

        lax.fori_loop(0, ACC // L, red_body, 0, unroll=2)

        for d in range(NUM_DOMAINS):
            cnt = tot_v[pl.ds(8 * D + d * L, L)]
            safe = jnp.maximum(cnt, 1.0)
            for j in range(D // L):
                off = d * D + j * L
                sm = tot_v[pl.ds(off, L)]
                sq = tot_v[pl.ds(4 * D + off, L)]
                mean = sm / safe
                var = jnp.maximum(sq / safe - mean * mean, 0.0)
                s = g_v[pl.ds(off, L)] * _rsqrt(var + EPS)
                so_v[pl.ds(off, L)] = s
                so_v[pl.ds(4 * D + off, L)] = b_v[pl.ds(off, L)] - mean * s

        pltpu.sync_copy(so_v, so_sh)

    plsc.subcore_barrier()
    pltpu.sync_copy(so_sh, so_v)

    def norm_body(r, _):
        d_b = plsc.load_gather(didx_v, [jnp.zeros((L,), jnp.int32) + r])
        col = d_b * D + iota
        vs = [data_v[pl.ds(r * D + j * L, L)] for j in range(D // L)]
        ss = [plsc.load_gather(so_v, [col + j * L]) for j in range(D // L)]
        os_ = [plsc.load_gather(so_v, [col + j * L + 4 * D])
               for j in range(D // L)]
        for j in range(D // L):
            data_v[pl.ds(r * D + j * L, L)] = vs[j] * ss[j] + os_[j]
        return 0

    ocps = []
    for c in range(CH):
        cps[c].wait()
        lax.fori_loop(c * CR, (c + 1) * CR, norm_body, 0, unroll=4)
        ocps.append(pltpu.async_copy(
            data_v.at[pl.ds(c * CR * D, CR * D)],
            out_hbm.at[pl.ds((row0 + c * CR) * D, CR * D)],
            osems[c]))
    for c in range(CH):
        ocps[c].wait()


def kernel(inputs, domain_indicator, gamma, beta):
    x = inputs.reshape(-1)
    di = domain_indicator.reshape(-1)
    parts, didx = _stats_kernel(x, di)
    out = _norm_kernel(x, didx, parts,
                       gamma[:, :D].reshape(-1), beta[:, :D].reshape(-1))
    return out.reshape(B, D)
